# baseline (speedup 1.0000x reference)
import functools

import jax
import jax.numpy as jnp
from jax import lax
from jax.experimental import pallas as pl
from jax.experimental.pallas import tpu as pltpu
from jax.experimental.pallas import tpu_sc as plsc

B = 16384
D = 64
NV = 1000000
NUM_CORES = 2
NUM_SUBCORES = 16
NW = NUM_CORES * NUM_SUBCORES
BPW = B // NW
CHUNK = 128
NCH = BPW // CHUNK
TC_BLK = 2048


def _gather_body(uids_hbm, iids_hbm, utab2_hbm, itab2_hbm,
                 uout2_hbm, iout2_hbm,
                 uidx_v, iidx_v, ubuf0, ibuf0, ubuf1, ibuf1,
                 sem_g0, sem_g1, sem_o):
    wid = lax.axis_index("s") * NUM_CORES + lax.axis_index("c")
    base = wid * BPW

    pltpu.sync_copy(uids_hbm.at[pl.ds(base, BPW)], uidx_v)
    pltpu.sync_copy(iids_hbm.at[pl.ds(base, BPW)], iidx_v)

    ubufs = (ubuf0, ubuf1)
    ibufs = (ibuf0, ibuf1)
    gsems = (sem_g0, sem_g1)

    def start_gather(c):
        slot = c % 2
        sl = pl.ds(c * CHUNK, CHUNK)
        cu = pltpu.async_copy(utab2_hbm.at[uidx_v.at[sl]], ubufs[slot], gsems[slot])
        ci = pltpu.async_copy(itab2_hbm.at[iidx_v.at[sl]], ibufs[slot], gsems[slot])
        return cu, ci

    pend = start_gather(0)
    prev_out = None
    for c in range(NCH):
        slot = c % 2
        cu, ci = pend
        cu.wait()
        ci.wait()
        if c + 1 < NCH:
            pend = start_gather(c + 1)
        if prev_out is not None:
            for cp in prev_out:
                cp.wait()
        hb = base + c * CHUNK
        prev_out = (
            pltpu.async_copy(ubufs[slot], uout2_hbm.at[pl.ds(hb, CHUNK)], sem_o),
            pltpu.async_copy(ibufs[slot], iout2_hbm.at[pl.ds(hb, CHUNK)], sem_o),
        )
    for cp in prev_out:
        cp.wait()


def _dot_body(u_ref, i_ref, o_ref):
    prod = u_ref[...] * i_ref[...]
    o_ref[...] = jnp.sum(prod, axis=1)


@jax.jit
def _run(user_ids, item_ids, user_table, item_table):
    utab2 = user_table[:NV].reshape(NV // 2, 2 * D)
    itab2 = item_table[:NV].reshape(NV // 2, 2 * D)
    upair_ids = lax.shift_right_logical(user_ids, 1)
    ipair_ids = lax.shift_right_logical(item_ids, 1)
    mesh = plsc.VectorSubcoreMesh(core_axis_name="c", subcore_axis_name="s")
    gather = functools.partial(
        pl.kernel,
        out_type=[
            jax.ShapeDtypeStruct((B, 2 * D), jnp.float32),
            jax.ShapeDtypeStruct((B, 2 * D), jnp.float32),
        ],
        mesh=mesh,
        compiler_params=pltpu.CompilerParams(needs_layout_passes=False),
        scratch_types=[
            pltpu.VMEM((BPW,), jnp.int32),
            pltpu.VMEM((BPW,), jnp.int32),
            pltpu.VMEM((CHUNK, 2 * D), jnp.float32),
            pltpu.VMEM((CHUNK, 2 * D), jnp.float32),
            pltpu.VMEM((CHUNK, 2 * D), jnp.float32),
            pltpu.VMEM((CHUNK, 2 * D), jnp.float32),
            pltpu.SemaphoreType.DMA,
            pltpu.SemaphoreType.DMA,
            pltpu.SemaphoreType.DMA,
        ],
    )(_gather_body)
    u_pair, i_pair = gather(upair_ids, ipair_ids, utab2, itab2)

    odd_u = (user_ids & 1).astype(jnp.bool_)[:, None]
    odd_i = (item_ids & 1).astype(jnp.bool_)[:, None]
    u_emb = jnp.where(odd_u, u_pair[:, D:], u_pair[:, :D])
    i_emb = jnp.where(odd_i, i_pair[:, D:], i_pair[:, :D])

    score = pl.pallas_call(
        _dot_body,
        grid=(B // TC_BLK,),
        in_specs=[
            pl.BlockSpec((TC_BLK, D), lambda g: (g, 0)),
            pl.BlockSpec((TC_BLK, D), lambda g: (g, 0)),
        ],
        out_specs=pl.BlockSpec((TC_BLK,), lambda g: (g,)),
        out_shape=jax.ShapeDtypeStruct((B,), jnp.float32),
    )(u_emb, i_emb)

    return score, u_emb, i_emb


def kernel(user_ids, item_ids, user_table, item_table):
    score, u_emb, i_emb = _run(
        user_ids.astype(jnp.int32), item_ids.astype(jnp.int32),
        user_table, item_table)
    b = user_ids.shape[0]
    return (score, u_emb.reshape(b, 1, D), i_emb.reshape(b, D, 1))


# speedup vs baseline: 1.0690x; 1.0690x over previous
import functools

import jax
import jax.numpy as jnp
from jax import lax
from jax.experimental import pallas as pl
from jax.experimental.pallas import tpu as pltpu
from jax.experimental.pallas import tpu_sc as plsc

B = 16384
D = 64
NV = 1000000
NUM_CORES = 2
NUM_SUBCORES = 16
NW = NUM_CORES * NUM_SUBCORES
BPW = B // NW
CHUNK = 128
NCH = BPW // CHUNK
TC_BLK = 2048


def _gather_body(uids_hbm, iids_hbm, utab2_hbm, itab2_hbm,
                 uout2_hbm, iout2_hbm,
                 uidx_v, iidx_v, ubuf0, ibuf0, ubuf1, ibuf1,
                 sem_g0, sem_g1, sem_o):
    wid = lax.axis_index("s") * NUM_CORES + lax.axis_index("c")
    base = wid * BPW

    pltpu.sync_copy(uids_hbm.at[pl.ds(base, BPW)], uidx_v)
    pltpu.sync_copy(iids_hbm.at[pl.ds(base, BPW)], iidx_v)

    ubufs = (ubuf0, ubuf1)
    ibufs = (ibuf0, ibuf1)
    gsems = (sem_g0, sem_g1)

    def start_gather(c):
        slot = c % 2
        sl = pl.ds(c * CHUNK, CHUNK)
        cu = pltpu.async_copy(utab2_hbm.at[uidx_v.at[sl]], ubufs[slot], gsems[slot])
        ci = pltpu.async_copy(itab2_hbm.at[iidx_v.at[sl]], ibufs[slot], gsems[slot])
        return cu, ci

    pend = start_gather(0)
    prev_out = None
    for c in range(NCH):
        slot = c % 2
        cu, ci = pend
        cu.wait()
        ci.wait()
        if c + 1 < NCH:
            pend = start_gather(c + 1)
        if prev_out is not None:
            for cp in prev_out:
                cp.wait()
        hb = base + c * CHUNK
        prev_out = (
            pltpu.async_copy(ubufs[slot], uout2_hbm.at[pl.ds(hb, CHUNK)], sem_o),
            pltpu.async_copy(ibufs[slot], iout2_hbm.at[pl.ds(hb, CHUNK)], sem_o),
        )
    for cp in prev_out:
        cp.wait()


def _dot_body(u_ref, i_ref, o_ref):
    prod = u_ref[...] * i_ref[...]
    o_ref[...] = jnp.sum(prod[:, :D], axis=1)


@jax.jit
def _run(user_ids, item_ids, user_table, item_table):
    utab2 = jnp.pad(user_table[:NV], ((0, 0), (0, D)))
    itab2 = jnp.pad(item_table[:NV], ((0, 0), (0, D)))
    mesh = plsc.VectorSubcoreMesh(core_axis_name="c", subcore_axis_name="s")
    gather = functools.partial(
        pl.kernel,
        out_type=[
            jax.ShapeDtypeStruct((B, 2 * D), jnp.float32),
            jax.ShapeDtypeStruct((B, 2 * D), jnp.float32),
        ],
        mesh=mesh,
        compiler_params=pltpu.CompilerParams(needs_layout_passes=False),
        scratch_types=[
            pltpu.VMEM((BPW,), jnp.int32),
            pltpu.VMEM((BPW,), jnp.int32),
            pltpu.VMEM((CHUNK, 2 * D), jnp.float32),
            pltpu.VMEM((CHUNK, 2 * D), jnp.float32),
            pltpu.VMEM((CHUNK, 2 * D), jnp.float32),
            pltpu.VMEM((CHUNK, 2 * D), jnp.float32),
            pltpu.SemaphoreType.DMA,
            pltpu.SemaphoreType.DMA,
            pltpu.SemaphoreType.DMA,
        ],
    )(_gather_body)
    u_pad, i_pad = gather(user_ids, item_ids, utab2, itab2)

    score = pl.pallas_call(
        _dot_body,
        grid=(B // TC_BLK,),
        in_specs=[
            pl.BlockSpec((TC_BLK, 2 * D), lambda g: (g, 0)),
            pl.BlockSpec((TC_BLK, 2 * D), lambda g: (g, 0)),
        ],
        out_specs=pl.BlockSpec((TC_BLK,), lambda g: (g,)),
        out_shape=jax.ShapeDtypeStruct((B,), jnp.float32),
    )(u_pad, i_pad)

    return score, u_pad[:, :D], i_pad[:, :D]


def kernel(user_ids, item_ids, user_table, item_table):
    score, u_emb, i_emb = _run(
        user_ids.astype(jnp.int32), item_ids.astype(jnp.int32),
        user_table, item_table)
    b = user_ids.shape[0]
    return (score, u_emb.reshape(b, 1, D), i_emb.reshape(b, D, 1))
